# Initial kernel scaffold; baseline (speedup 1.0000x reference)
import jax
import jax.numpy as jnp
from jax.experimental import pallas as pl


def kernel(x, t, weight_vals, t_weights):
    raise NotImplementedError("write your pallas kernel here")


# speedup vs baseline: 1.5936x; 1.5936x over previous
import jax
import jax.numpy as jnp
from jax import lax
from jax.experimental import pallas as pl
from jax.experimental.pallas import tpu as pltpu
from jax.experimental.pallas import tpu_sc as plsc

_NUM_IN = 8192
_NUM_OUT = 2048
_BF = 4
_BATCH = 4096
_L = 16
_NC = 2
_NS = 16
_NW = _NC * _NS
_ROWS = _BATCH // _NW
_R = 8
_NCHUNK = _ROWS // _R
_OG = _NUM_OUT // _L


def _sc_body(x_hbm, tf_hbm, w_hbm, tw_hbm, out_hbm,
             x_tile, tf_tile, w_tile, tw_tile, out_tile):
    wid = lax.axis_index("s") * _NC + lax.axis_index("c")
    base = wid * _ROWS
    pltpu.sync_copy(w_hbm, w_tile)
    pltpu.sync_copy(tw_hbm, tw_tile)
    pltpu.sync_copy(tf_hbm.at[pl.ds(base, _ROWS)], tf_tile)
    lane4 = lax.broadcasted_iota(jnp.int32, (_L,), 0) * _BF

    def chunk_body(ci, carry):
        r0 = base + ci * _R
        pltpu.sync_copy(x_hbm.at[pl.ds(r0, _R)], x_tile)
        tbs = [plsc.load_gather(tf_tile, [jnp.full((_L,), ci * _R + r, jnp.int32)])
               for r in range(_R)]

        def ogrp_body(g, carry2):
            o0 = g * _L
            tw_v = tw_tile[pl.ds(o0, _L)]
            w_vs = [w_tile[j, pl.ds(o0, _L)] for j in range(_BF)]
            cbase = lane4 + o0 * _BF
            for r in range(_R):
                ridx = jnp.full((_L,), r, jnp.int32)
                acc = tw_v * tbs[r]
                for j in range(_BF):
                    xv = plsc.load_gather(x_tile, [ridx, cbase + j])
                    acc = acc + w_vs[j] * xv
                out_tile[r, pl.ds(o0, _L)] = acc
            return carry2

        lax.fori_loop(0, _OG, ogrp_body, 0)
        pltpu.sync_copy(out_tile, out_hbm.at[pl.ds(r0, _R)])
        return carry

    lax.fori_loop(0, _NCHUNK, chunk_body, 0)


def kernel(x, t, weight_vals, t_weights):
    tf = t.astype(jnp.float32)
    w4 = weight_vals.reshape(_NUM_OUT, _BF).T
    tw = t_weights.reshape(_NUM_OUT)
    mesh = plsc.VectorSubcoreMesh(core_axis_name="c", subcore_axis_name="s")
    f = pl.kernel(
        _sc_body,
        out_type=jax.ShapeDtypeStruct((_BATCH, _NUM_OUT), jnp.float32),
        mesh=mesh,
        scratch_types=[
            pltpu.VMEM((_R, _NUM_IN), jnp.float32),
            pltpu.VMEM((_ROWS,), jnp.float32),
            pltpu.VMEM((_BF, _NUM_OUT), jnp.float32),
            pltpu.VMEM((_NUM_OUT,), jnp.float32),
            pltpu.VMEM((_R, _NUM_OUT), jnp.float32),
        ],
        compiler_params=pltpu.CompilerParams(needs_layout_passes=False),
    )
    return f(x, tf, w4, tw)
